# baseline (speedup 1.0000x reference)
import functools

import jax
import jax.numpy as jnp
from jax import lax
from jax.experimental import pallas as pl
from jax.experimental.pallas import tpu as pltpu
from jax.experimental.pallas import tpu_sc as plsc

NTOKEN = 1000000
EMB_DIM = 64

_info = plsc.get_sparse_core_info()
_NC, _NS = _info.num_cores, _info.num_subcores
_NW = _NC * _NS

_NB = 16384
_NJ = 20
_BLK = 128
_BPW = _NB // _NW
_NBT = _BPW // _BLK
_NBLK = _NJ * _NBT


def _make_kernel():
    mesh = plsc.VectorSubcoreMesh(core_axis_name="c", subcore_axis_name="s")

    @functools.partial(
        pl.kernel,
        mesh=mesh,
        out_type=jax.ShapeDtypeStruct((_NJ, EMB_DIM, _NB), jnp.float32),
        scratch_types=[
            pltpu.VMEM((_NJ, _BPW), jnp.int32),
            pltpu.VMEM((2, _BLK, EMB_DIM), jnp.float32),
            pltpu.VMEM((2, 8, 8, _BLK), jnp.float32),
            pltpu.SemaphoreType.DMA((2,)),
            pltpu.SemaphoreType.DMA((2,)),
        ],
        compiler_params=pltpu.CompilerParams(use_tc_tiling_on_sc=True,
                                              needs_layout_passes=False),
    )
    def emb_kernel(table_hbm, xt_hbm, out_hbm, idx_v, gbuf, tbuf, gsem, wsem):
        wid = lax.axis_index("s") * _NC + lax.axis_index("c")
        col0 = wid * _BPW
        pltpu.sync_copy(xt_hbm.at[:, pl.ds(col0, _BPW)], idx_v)

        iotas = [jax.lax.broadcasted_iota(jnp.int32, (16,), 0) + 16 * g
                 for g in range(8)]

        def fire(k, p):
            j, bti = k // _NBT, k % _NBT
            for g in range(8):
                iv = idx_v[j, pl.ds(bti * _BLK + g * 16, 16)]
                for u in range(16):
                    pltpu.async_copy(table_hbm.at[pl.ds(iv[u], 1)],
                                     gbuf.at[p, pl.ds(g * 16 + u, 1)],
                                     gsem.at[p])

        def wait_gather(p):
            pltpu.make_async_copy(table_hbm.at[pl.ds(0, _BLK)],
                                  gbuf.at[p], gsem.at[p]).wait()

        def wait_writes(p):
            for ft in range(8):
                pltpu.make_async_copy(
                    out_hbm.at[0, pl.ds(ft * 8, 8), pl.ds(0, _BLK)],
                    tbuf.at[p, ft], wsem.at[p]).wait()

        def transpose_and_write(k, p):
            j, bti = k // _NBT, k % _NBT
            g2 = gbuf.at[p]
            for ft in range(8):
                for fs in range(8):
                    col = jnp.full((16,), ft * 8 + fs, jnp.int32)
                    for g in range(8):
                        v = plsc.load_gather(g2, [iotas[g], col])
                        tbuf[p, ft, fs, pl.ds(16 * g, 16)] = v
            for ft in range(8):
                pltpu.async_copy(
                    tbuf.at[p, ft],
                    out_hbm.at[j, pl.ds(ft * 8, 8),
                               pl.ds(col0 + bti * _BLK, _BLK)],
                    wsem.at[p])

        fire(0, 0)
        fire(1, 1)

        def step(k2, _):
            for p in range(2):
                k = k2 * 2 + p
                wait_gather(p)

                @pl.when(k >= 2)
                def _drain():
                    wait_writes(p)

                transpose_and_write(k, p)

                @pl.when(k + 2 < _NBLK)
                def _refill():
                    fire(k + 2, p)
            return _

        lax.fori_loop(0, _NBLK // 2, step, None)
        wait_writes(0)
        wait_writes(1)

    return emb_kernel


_emb_kernel = _make_kernel()


@jax.jit
def kernel(x, table):
    xt = x.astype(jnp.int32).T
    out_t = _emb_kernel(table, xt)
    return out_t.transpose(2, 0, 1)


# speedup vs baseline: 1.5775x; 1.5775x over previous
import functools

import jax
import jax.numpy as jnp
from jax import lax
from jax.experimental import pallas as pl
from jax.experimental.pallas import tpu as pltpu
from jax.experimental.pallas import tpu_sc as plsc

NTOKEN = 1000000
EMB_DIM = 64

_info = plsc.get_sparse_core_info()
_NC, _NS = _info.num_cores, _info.num_subcores
_NW = _NC * _NS

_NB = 16384
_NJ = 20
_B = _NB * _NJ
_BPW = _B // _NW
_CB = 16
_C = _CB * _NJ
_NCHUNK = _BPW // _C
_NBUF = 2
_NG = _NCHUNK // _NBUF


def _make_kernel():
    mesh = plsc.VectorSubcoreMesh(core_axis_name="c", subcore_axis_name="s")

    @functools.partial(
        pl.kernel,
        mesh=mesh,
        out_type=jax.ShapeDtypeStruct((_NB, _NJ, EMB_DIM), jnp.float32),
        scratch_types=[
            pltpu.VMEM((_NBUF, _C), jnp.int32),
            pltpu.VMEM((_NBUF, _CB, _NJ, EMB_DIM), jnp.float32),
            pltpu.SemaphoreType.DMA((_NBUF,)),
        ],
        compiler_params=pltpu.CompilerParams(use_tc_tiling_on_sc=True),
    )
    def emb_kernel(table_hbm, idx_hbm, out_hbm, idx_v, rows_v, gsem):
        wid = lax.axis_index("s") * _NC + lax.axis_index("c")
        base = wid * (_BPW // _NJ)

        def fire(t, b):
            pltpu.sync_copy(idx_hbm.at[wid, t], idx_v.at[b])

            def row16(q, _):
                r = q * 16
                iv = idx_v[b, pl.ds(r, 16)]
                for u in range(16):
                    rr = r + u
                    pltpu.async_copy(table_hbm.at[pl.ds(iv[u], 1)],
                                     rows_v.at[b, rr // _NJ,
                                               pl.ds(rr % _NJ, 1)],
                                     gsem.at[b])
                return _

            lax.fori_loop(0, _C // 16, row16, None)

        for b in range(_NBUF):
            fire(b, b)

        def group(g, _):
            for b in range(_NBUF):
                t = g * _NBUF + b
                pltpu.make_async_copy(out_hbm.at[pl.ds(0, _CB)],
                                      rows_v.at[b], gsem.at[b]).wait()
                pltpu.sync_copy(rows_v.at[b],
                                out_hbm.at[pl.ds(base + t * _CB, _CB)])

                @pl.when(g < _NG - 1)
                def _refill():
                    fire(t + _NBUF, b)
            return _

        lax.fori_loop(0, _NG, group, None)

    return emb_kernel


_emb_kernel = _make_kernel()


@jax.jit
def kernel(x, table):
    idx = x.astype(jnp.int32).reshape(_NW, _NCHUNK, _C)
    return _emb_kernel(table, idx)
